# baseline (speedup 1.0000x reference)
import functools

import jax
import jax.numpy as jnp
from jax import lax
from jax.experimental import pallas as pl
from jax.experimental.pallas import tpu as pltpu
from jax.experimental.pallas import tpu_sc as plsc

L = 16

_OW1 = 0
_OB1 = _OW1 + 128 * 32
_OW2 = _OB1 + 32
_OB2 = _OW2 + 32 * 32
_OW3 = _OB2 + 32
_OB3 = _OW3 + 32 * 64
_WTOT = _OB3 + 64


def _body(idx_hbm, tableT_hbm, w_hbm, out_hbm,
          idx_v, blk0_v, blk1_v, tail_v, w_v, out_v,
          sem_w, sem_g0, sem_g1, sem_t):
  cid = lax.axis_index("c")
  sid = lax.axis_index("s")
  V = tableT_hbm.shape[1]
  tail = (V // 128) * 128
  last_full = tail - 128

  @pl.when((cid == 0) & (sid == 0))
  def _():
    wcp = pltpu.async_copy(w_hbm, w_v, sem_w)
    tcp = pltpu.async_copy(tableT_hbm.at[:, pl.ds(tail, V - tail)], tail_v,
                           sem_t)
    pltpu.sync_copy(idx_hbm, idx_v)
    iv = idx_v[pl.ds(0, L)]
    i0 = iv[0]
    i1 = iv[1]
    a0 = pl.multiple_of(jnp.minimum((i0 // 128) * 128, last_full), 128)
    a1 = pl.multiple_of(jnp.minimum((i1 // 128) * 128, last_full), 128)
    g0 = pltpu.async_copy(tableT_hbm.at[:, pl.ds(a0, 128)], blk0_v, sem_g0)
    g1 = pltpu.async_copy(tableT_hbm.at[:, pl.ds(a1, 128)], blk1_v, sem_g1)
    g0.wait()
    g1.wait()
    tcp.wait()
    wcp.wait()

    lanes = lax.broadcasted_iota(jnp.int32, (L,), 0)

    def column(blk, tidx, i, a):
      in_main = i < tail
      lane_m = jnp.full((L,), jnp.minimum(i - a, 127), dtype=jnp.int32)
      lane_t = jnp.full((L,), jnp.clip(i - tail, 0, V - tail - 1),
                        dtype=jnp.int32)
      sel = jnp.full((L,), in_main)
      return [jnp.where(sel,
                        plsc.load_gather(blk, [c * L + lanes, lane_m]),
                        plsc.load_gather(tidx, [c * L + lanes, lane_t]))
              for c in range(4)]

    xv = column(blk0_v, tail_v, i0, a0) + column(blk1_v, tail_v, i1, a1)

    acc1 = [w_v[pl.ds(_OB1 + c * L, L)] for c in range(2)]
    for k in range(128):
      s = xv[k // L][k % L]
      for c in range(2):
        acc1[c] = acc1[c] + s * w_v[pl.ds(_OW1 + k * 32 + c * L, L)]
    h1 = [jnp.maximum(a, 0.0) for a in acc1]

    acc2 = [w_v[pl.ds(_OB2 + c * L, L)] for c in range(2)]
    for k in range(32):
      s = h1[k // L][k % L]
      for c in range(2):
        acc2[c] = acc2[c] + s * w_v[pl.ds(_OW2 + k * 32 + c * L, L)]
    h2 = [jnp.maximum(a, 0.0) for a in acc2]

    acc3 = [w_v[pl.ds(_OB3 + c * L, L)] for c in range(4)]
    for k in range(32):
      s = h2[k // L][k % L]
      for c in range(4):
        acc3[c] = acc3[c] + s * w_v[pl.ds(_OW3 + k * 64 + c * L, L)]
    for c in range(4):
      out_v[0, pl.ds(c * L, L)] = jnp.maximum(acc3[c], 0.0)

    pltpu.sync_copy(out_v, out_hbm)


@jax.jit
def _run(idx16, tableT, w_flat):
  mesh = plsc.VectorSubcoreMesh(
      core_axis_name="c", subcore_axis_name="s", num_cores=2, num_subcores=16)
  return pl.kernel(
      _body,
      out_type=jax.ShapeDtypeStruct((1, 64), jnp.float32),
      mesh=mesh,
      scratch_types=[
          pltpu.VMEM((L,), jnp.int32),
          pltpu.VMEM((64, 128), jnp.float32),
          pltpu.VMEM((64, 128), jnp.float32),
          pltpu.VMEM((64, 64), jnp.float32),
          pltpu.VMEM((_WTOT,), jnp.float32),
          pltpu.VMEM((1, 64), jnp.float32),
          pltpu.SemaphoreType.DMA,
          pltpu.SemaphoreType.DMA,
          pltpu.SemaphoreType.DMA,
          pltpu.SemaphoreType.DMA,
      ],
      compiler_params=pltpu.CompilerParams(needs_layout_passes=False),
  )(idx16, tableT, w_flat)


def kernel(inputs, table, W1, b1, W2, b2, W3, b3):
  idx16 = jnp.zeros((L,), jnp.int32).at[:2].set(inputs.astype(jnp.int32))
  w_flat = jnp.concatenate([
      W1.T.reshape(-1), b1, W2.T.reshape(-1), b2, W3.T.reshape(-1), b3])
  return _run(idx16, table.T, w_flat)


# speedup vs baseline: 1.0539x; 1.0539x over previous
import functools

import jax
import jax.numpy as jnp
from jax import lax
from jax.experimental import pallas as pl
from jax.experimental.pallas import tpu as pltpu
from jax.experimental.pallas import tpu_sc as plsc

L = 16


def _body(idx_hbm, tableT_hbm, w1t_hbm, b1_hbm, w2t_hbm, b2_hbm,
          w3t_hbm, b3_hbm, out_hbm,
          idx_v, blk0_v, blk1_v, tail_v, w1t_v, b1_v, w2t_v, b2_v,
          w3t_v, b3_v, out_v, sem_w, sem_g0, sem_g1):
  cid = lax.axis_index("c")
  sid = lax.axis_index("s")
  V = tableT_hbm.shape[1]
  tail = (V // 128) * 128
  last_full = tail - 128

  @pl.when((cid == 0) & (sid == 0))
  def _():
    wcps = [pltpu.async_copy(src, dst, sem_w) for src, dst in (
        (w1t_hbm, w1t_v), (b1_hbm, b1_v), (w2t_hbm, w2t_v), (b2_hbm, b2_v),
        (w3t_hbm, w3t_v), (b3_hbm, b3_v),
        (tableT_hbm.at[:, pl.ds(tail, V - tail)], tail_v),
    )]
    pltpu.sync_copy(idx_hbm, idx_v.at[pl.ds(0, 2)])
    iv = idx_v[pl.ds(0, L)]
    i0 = iv[0]
    i1 = iv[1]
    a0 = pl.multiple_of(jnp.minimum((i0 // 128) * 128, last_full), 128)
    a1 = pl.multiple_of(jnp.minimum((i1 // 128) * 128, last_full), 128)
    g0 = pltpu.async_copy(tableT_hbm.at[:, pl.ds(a0, 128)], blk0_v, sem_g0)
    g1 = pltpu.async_copy(tableT_hbm.at[:, pl.ds(a1, 128)], blk1_v, sem_g1)
    g0.wait()
    g1.wait()
    for cp in wcps:
      cp.wait()

    lanes = lax.broadcasted_iota(jnp.int32, (L,), 0)

    def column(blk, i, a):
      in_main = i < tail
      lane_m = jnp.full((L,), jnp.minimum(i - a, 127), dtype=jnp.int32)
      lane_t = jnp.full((L,), jnp.clip(i - tail, 0, V - tail - 1),
                        dtype=jnp.int32)
      sel = jnp.full((L,), in_main)
      return [jnp.where(sel,
                        plsc.load_gather(blk, [c * L + lanes, lane_m]),
                        plsc.load_gather(tail_v, [c * L + lanes, lane_t]))
              for c in range(4)]

    xv = column(blk0_v, i0, a0) + column(blk1_v, i1, a1)

    acc1 = [b1_v[pl.ds(c * L, L)] for c in range(2)]
    for k in range(128):
      s = xv[k // L][k % L]
      for c in range(2):
        acc1[c] = acc1[c] + s * w1t_v[k, pl.ds(c * L, L)]
    h1 = [jnp.maximum(a, 0.0) for a in acc1]

    acc2 = [b2_v[pl.ds(c * L, L)] for c in range(2)]
    for k in range(32):
      s = h1[k // L][k % L]
      for c in range(2):
        acc2[c] = acc2[c] + s * w2t_v[k, pl.ds(c * L, L)]
    h2 = [jnp.maximum(a, 0.0) for a in acc2]

    acc3 = [b3_v[pl.ds(c * L, L)] for c in range(4)]
    for k in range(32):
      s = h2[k // L][k % L]
      for c in range(4):
        acc3[c] = acc3[c] + s * w3t_v[k, pl.ds(c * L, L)]
    for c in range(4):
      out_v[0, pl.ds(c * L, L)] = jnp.maximum(acc3[c], 0.0)

    pltpu.sync_copy(out_v, out_hbm)


@jax.jit
def _run(idx, tableT, w1t, b1, w2t, b2, w3t, b3):
  mesh = plsc.VectorSubcoreMesh(
      core_axis_name="c", subcore_axis_name="s", num_cores=2, num_subcores=16)
  return pl.kernel(
      _body,
      out_type=jax.ShapeDtypeStruct((1, 64), jnp.float32),
      mesh=mesh,
      scratch_types=[
          pltpu.VMEM((L,), jnp.int32),
          pltpu.VMEM((64, 128), jnp.float32),
          pltpu.VMEM((64, 128), jnp.float32),
          pltpu.VMEM((64, 64), jnp.float32),
          pltpu.VMEM((128, 32), jnp.float32),
          pltpu.VMEM((32,), jnp.float32),
          pltpu.VMEM((32, 32), jnp.float32),
          pltpu.VMEM((32,), jnp.float32),
          pltpu.VMEM((32, 64), jnp.float32),
          pltpu.VMEM((64,), jnp.float32),
          pltpu.VMEM((1, 64), jnp.float32),
          pltpu.SemaphoreType.DMA,
          pltpu.SemaphoreType.DMA,
          pltpu.SemaphoreType.DMA,
      ],
      compiler_params=pltpu.CompilerParams(needs_layout_passes=False),
  )(idx, tableT, w1t, b1, w2t, b2, w3t, b3)


def kernel(inputs, table, W1, b1, W2, b2, W3, b3):
  return _run(inputs.astype(jnp.int32), table.T, W1.T, b1, W2.T, b2, W3.T, b3)
